# baseline (speedup 1.0000x reference)
import jax
import jax.numpy as jnp
from jax.experimental import pallas as pl
from jax.experimental.pallas import tpu as pltpu


def _max_kernel(x_ref, o_ref):
    o_ref[:, 0] = jnp.max(x_ref[...], axis=-1)


def kernel(X):
    out = pl.pallas_call(
        _max_kernel,
        grid=(16,),
        in_specs=[pl.BlockSpec((8, 100000), lambda i: (i, 0))],
        out_specs=pl.BlockSpec((8, 1), lambda i: (i, 0)),
        out_shape=jax.ShapeDtypeStruct((128, 1), jnp.float32),
        compiler_params=pltpu.CompilerParams(
            dimension_semantics=("parallel",),
        ),
    )(X)
    return out[:, 0]
